# Initial kernel scaffold; baseline (speedup 1.0000x reference)
import jax
import jax.numpy as jnp
from jax.experimental import pallas as pl


def kernel(inputs, edge_index0, edge_index1, W0, W1, dropout):
    raise NotImplementedError("write your pallas kernel here")


# speedup vs baseline: 4.7382x; 4.7382x over previous
import functools

import jax
import jax.numpy as jnp
from jax import lax
from jax.experimental import pallas as pl
from jax.experimental.pallas import tpu as pltpu
from jax.experimental.pallas import tpu_sc as plsc

N = 10000
E = 320000
D = 128

NS = 16
EPT = E // NS
K = 80
NCHUNK = EPT // K
NPAD = 10240
RPT = NPAD // NS
ZR = 128
ZSTEPS = RPT // ZR

_MM_BLOCK = 1000


def _mm_body(x_ref, w0_ref, w1_ref, o0_ref, o1_ref):
    x = x_ref[...]
    o0_ref[...] = jnp.dot(x, w0_ref[...], preferred_element_type=jnp.float32)
    o1_ref[...] = jnp.dot(x, w1_ref[...], preferred_element_type=jnp.float32)


def _matmul(x, W0, W1):
    grid = (N // _MM_BLOCK,)
    return pl.pallas_call(
        _mm_body,
        grid=grid,
        in_specs=[
            pl.BlockSpec((_MM_BLOCK, D), lambda i: (i, 0)),
            pl.BlockSpec((D, D), lambda i: (0, 0)),
            pl.BlockSpec((D, D), lambda i: (0, 0)),
        ],
        out_specs=[
            pl.BlockSpec((_MM_BLOCK, D), lambda i: (i, 0)),
            pl.BlockSpec((_MM_BLOCK, D), lambda i: (i, 0)),
        ],
        out_shape=[
            jax.ShapeDtypeStruct((N, D), jnp.float32),
            jax.ShapeDtypeStruct((N, D), jnp.float32),
        ],
    )(x, W0, W1)


def _epi_body(a0_ref, a1_ref, o_ref):
    o = jnp.maximum(a0_ref[...], 0.0) + jnp.maximum(a1_ref[...], 0.0)
    ss = jnp.sum(o * o, axis=1, keepdims=True)
    norm = jnp.maximum(jnp.sqrt(ss), 1e-12)
    o_ref[...] = o / norm


def _epilogue(a0, a1):
    blk = 1024
    grid = (NPAD // blk,)
    return pl.pallas_call(
        _epi_body,
        grid=grid,
        in_specs=[
            pl.BlockSpec((blk, D), lambda i: (i, 0)),
            pl.BlockSpec((blk, D), lambda i: (i, 0)),
        ],
        out_specs=pl.BlockSpec((blk, D), lambda i: (i, 0)),
        out_shape=jax.ShapeDtypeStruct((NPAD, D), jnp.float32),
    )(a0, a1)


_sc_mesh = plsc.VectorSubcoreMesh(core_axis_name="c", subcore_axis_name="s")


@functools.partial(
    pl.kernel,
    out_type=(
        jax.ShapeDtypeStruct((NPAD, D), jnp.float32),
        jax.ShapeDtypeStruct((NPAD, D), jnp.float32),
    ),
    mesh=_sc_mesh,
    scratch_types=[
        pltpu.VMEM_SHARED((NPAD, D), jnp.float32),
        pltpu.VMEM((ZR, D), jnp.float32),
        pltpu.VMEM((K,), jnp.int32),
        pltpu.VMEM((K,), jnp.int32),
        pltpu.VMEM((K, D), jnp.float32),
        pltpu.SemaphoreType.DMA,
    ],
)
def _sc_message_pass(xw0, src0, dst0, xw1, src1, dst1, out0, out1,
                     acc, stage, src_v, dst_v, rows_v, sem):
    c = lax.axis_index("c")
    s = lax.axis_index("s")

    def _zero_body(i, carry):
        r = i // (D // 16)
        j = i % (D // 16)
        stage[r, pl.ds(j * 16, 16)] = jnp.zeros((16,), jnp.float32)
        return carry

    lax.fori_loop(0, ZR * (D // 16), _zero_body, 0)

    for t in range(ZSTEPS):
        pltpu.sync_copy(stage, acc.at[pl.ds(s * RPT + t * ZR, ZR)])
    plsc.subcore_barrier()

    def _run_relation(xw, srcs, dsts, out):
        def _chunk(g, carry):
            e0 = s * EPT + g * K
            pltpu.sync_copy(srcs.at[pl.ds(e0, K)], src_v)
            pltpu.sync_copy(dsts.at[pl.ds(e0, K)], dst_v)
            pltpu.async_copy(xw.at[src_v], rows_v, sem).wait()
            pltpu.sync_copy(rows_v, acc.at[dst_v], add=True)
            return carry

        lax.fori_loop(0, NCHUNK, _chunk, 0)
        plsc.subcore_barrier()
        for t in range(ZSTEPS):
            base = s * RPT + t * ZR
            pltpu.sync_copy(acc.at[pl.ds(base, ZR)], stage)
            pltpu.sync_copy(stage, out.at[pl.ds(base, ZR)])

    @pl.when(c == 0)
    def _():
        _run_relation(xw0, src0, dst0, out0)

    @pl.when(c == 1)
    def _():
        _run_relation(xw1, src1, dst1, out1)


def kernel(inputs, edge_index0, edge_index1, W0, W1, dropout):
    xw0, xw1 = _matmul(inputs, W0, W1)
    acc0, acc1 = _sc_message_pass(
        xw0, edge_index0[0], edge_index0[1],
        xw1, edge_index1[0], edge_index1[1],
    )
    return _epilogue(acc0, acc1)[:N]


# speedup vs baseline: 10.9138x; 2.3034x over previous
import functools

import jax
import jax.numpy as jnp
from jax import lax
from jax.experimental import pallas as pl
from jax.experimental.pallas import tpu as pltpu
from jax.experimental.pallas import tpu_sc as plsc

N = 10000
E = 320000
D = 128

NS = 16
K = 128
CPT = 160
NCH_SLAB = 4
NSLAB = CPT // NCH_SLAB
NPAIR = NSLAB // 2
EPAD = NS * CPT * K
NPAD = 10240
RPT = NPAD // NS
ZR = 128
ZSTEPS = RPT // ZR
IDX_SRC, IDX_DST = 0, 1

_MM_BLOCK = 1000


def _mm_body(x_ref, w0_ref, w1_ref, o0_ref, o1_ref):
    x = x_ref[...]
    o0_ref[...] = jnp.dot(x, w0_ref[...], preferred_element_type=jnp.float32)
    o1_ref[...] = jnp.dot(x, w1_ref[...], preferred_element_type=jnp.float32)


def _matmul(x, W0, W1):
    grid = (N // _MM_BLOCK,)
    return pl.pallas_call(
        _mm_body,
        grid=grid,
        in_specs=[
            pl.BlockSpec((_MM_BLOCK, D), lambda i: (i, 0)),
            pl.BlockSpec((D, D), lambda i: (0, 0)),
            pl.BlockSpec((D, D), lambda i: (0, 0)),
        ],
        out_specs=[
            pl.BlockSpec((_MM_BLOCK, D), lambda i: (i, 0)),
            pl.BlockSpec((_MM_BLOCK, D), lambda i: (i, 0)),
        ],
        out_shape=[
            jax.ShapeDtypeStruct((N, D), jnp.float32),
            jax.ShapeDtypeStruct((N, D), jnp.float32),
        ],
    )(x, W0, W1)


def _epi_body(a0_ref, a1_ref, o_ref):
    o = jnp.maximum(a0_ref[...], 0.0) + jnp.maximum(a1_ref[...], 0.0)
    ss = jnp.sum(o * o, axis=1, keepdims=True)
    norm = jnp.maximum(jnp.sqrt(ss), 1e-12)
    o_ref[...] = o / norm


def _epilogue(a0, a1):
    blk = 1024
    grid = (NPAD // blk,)
    return pl.pallas_call(
        _epi_body,
        grid=grid,
        in_specs=[
            pl.BlockSpec((blk, D), lambda i: (i, 0)),
            pl.BlockSpec((blk, D), lambda i: (i, 0)),
        ],
        out_specs=pl.BlockSpec((blk, D), lambda i: (i, 0)),
        out_shape=jax.ShapeDtypeStruct((NPAD, D), jnp.float32),
    )(a0, a1)


_sc_mesh = plsc.VectorSubcoreMesh(core_axis_name="c", subcore_axis_name="s")


@functools.partial(
    pl.kernel,
    out_type=(
        jax.ShapeDtypeStruct((NPAD, D), jnp.float32),
        jax.ShapeDtypeStruct((NPAD, D), jnp.float32),
    ),
    mesh=_sc_mesh,
    scratch_types=[
        pltpu.VMEM_SHARED((NPAD, D), jnp.float32),
        [pltpu.VMEM((K, D), jnp.float32) for _ in range(2)],
        [pltpu.VMEM((2, NCH_SLAB, K), jnp.int32) for _ in range(2)],
        [pltpu.SemaphoreType.DMA for _ in range(2)],
        [pltpu.SemaphoreType.DMA for _ in range(2)],
        [pltpu.SemaphoreType.DMA for _ in range(2)],
    ],
)
def _sc_message_pass(xw0, idx0, xw1, idx1, out0, out1,
                     acc, rows, ibufs, gsems, ssems, isems):
    c = lax.axis_index("c")
    s = lax.axis_index("s")

    def _zero_body(i, carry):
        r = i // (D // 16)
        j = i % (D // 16)
        rows[0][r, pl.ds(j * 16, 16)] = jnp.zeros((16,), jnp.float32)
        return carry

    lax.fori_loop(0, ZR * (D // 16), _zero_body, 0)

    for t in range(ZSTEPS):
        pltpu.sync_copy(rows[0], acc.at[pl.ds(s * RPT + t * ZR, ZR)])
    plsc.subcore_barrier()

    def _run_relation(xw, idx, out):
        def _gather(p, j, b):
            pltpu.async_copy(xw.at[ibufs[p].at[IDX_SRC].at[j]], rows[b],
                             gsems[b])

        def _wait_gather(p, j, b):
            pltpu.make_async_copy(xw.at[ibufs[p].at[IDX_SRC].at[j]], rows[b],
                                  gsems[b]).wait()

        def _scatter(p, j, b):
            pltpu.async_copy(rows[b], acc.at[ibufs[p].at[IDX_DST].at[j]],
                             ssems[b], add=True)

        def _wait_scatter(p, j, b):
            pltpu.make_async_copy(rows[b], acc.at[ibufs[p].at[IDX_DST].at[j]],
                                  ssems[b]).wait()

        def _prefetch(slab, p):
            pltpu.async_copy(idx.at[s].at[slab], ibufs[p], isems[p])

        def _wait_prefetch(slab, p):
            pltpu.make_async_copy(idx.at[s].at[slab], ibufs[p],
                                  isems[p]).wait()

        _prefetch(0, 0)

        def _body(i, carry):
            nz = i > 0

            @pl.when(nz)
            def _():
                _wait_gather(1, NCH_SLAB - 1, 1)
                _scatter(1, NCH_SLAB - 1, 1)

            _wait_prefetch(2 * i, 0)

            @pl.when(nz)
            def _():
                _wait_scatter(0, 0, 0)
            _gather(0, 0, 0)
            @pl.when(nz)
            def _():
                _wait_scatter(0, 0, 1)
            _gather(0, 1, 1)
            _prefetch(2 * i + 1, 1)

            _wait_gather(0, 0, 0)
            _scatter(0, 0, 0)
            _wait_scatter(0, 0, 0)
            _gather(0, 2, 0)
            _wait_gather(0, 1, 1)
            _scatter(0, 1, 1)
            _wait_scatter(0, 1, 1)
            _gather(0, 3, 1)
            _wait_gather(0, 2, 0)
            _scatter(0, 2, 0)

            _wait_gather(0, 3, 1)
            _scatter(0, 3, 1)
            _wait_prefetch(2 * i + 1, 1)
            _wait_scatter(0, 2, 0)
            _gather(1, 0, 0)
            _wait_scatter(0, 3, 1)
            _gather(1, 1, 1)
            nxt = jnp.minimum(2 * i + 2, NSLAB - 1)
            _prefetch(nxt, 0)

            _wait_gather(1, 0, 0)
            _scatter(1, 0, 0)
            _wait_scatter(1, 0, 0)
            _gather(1, 2, 0)
            _wait_gather(1, 1, 1)
            _scatter(1, 1, 1)
            _wait_scatter(1, 1, 1)
            _gather(1, 3, 1)
            _wait_gather(1, 2, 0)
            _scatter(1, 2, 0)
            return carry

        lax.fori_loop(0, NPAIR, _body, 0)

        _wait_gather(1, NCH_SLAB - 1, 1)
        _scatter(1, NCH_SLAB - 1, 1)
        _wait_scatter(0, 0, 0)
        _wait_scatter(0, 0, 1)
        _wait_prefetch(NSLAB - 1, 0)

        plsc.subcore_barrier()
        for t in range(ZSTEPS):
            base = s * RPT + t * ZR
            pltpu.sync_copy(acc.at[pl.ds(base, ZR)], rows[t % 2])
            pltpu.sync_copy(rows[t % 2], out.at[pl.ds(base, ZR)])

    @pl.when(c == 0)
    def _():
        _run_relation(xw0, idx0, out0)

    @pl.when(c == 1)
    def _():
        _run_relation(xw1, idx1, out1)


def _pad_edges(ei):
    pad = EPAD - E
    src = jnp.concatenate(
        [ei[0], (jnp.arange(pad, dtype=jnp.int32) * 97) % N])
    dst = jnp.concatenate(
        [ei[1], N + (jnp.arange(pad, dtype=jnp.int32) % (NPAD - N))])
    src = src.reshape(NS, NSLAB, NCH_SLAB, K)
    dst = dst.reshape(NS, NSLAB, NCH_SLAB, K)
    return jnp.stack([src, dst], axis=2)


def kernel(inputs, edge_index0, edge_index1, W0, W1, dropout):
    xw0, xw1 = _matmul(inputs, W0, W1)
    idx0 = _pad_edges(edge_index0)
    idx1 = _pad_edges(edge_index1)
    acc0, acc1 = _sc_message_pass(xw0, idx0, xw1, idx1)
    return _epilogue(acc0, acc1)[:N]


# speedup vs baseline: 11.1047x; 1.0175x over previous
import functools

import jax
import jax.numpy as jnp
from jax import lax
from jax.experimental import pallas as pl
from jax.experimental.pallas import tpu as pltpu
from jax.experimental.pallas import tpu_sc as plsc

N = 10000
E = 320000
D = 128

NS = 16
K = 128
CPT = 160
NCH_SLAB = 4
NSLAB = CPT // NCH_SLAB
NPAIR = NSLAB // 2
EPAD = NS * CPT * K
NPAD = 10240
RPT = NPAD // NS
ZR = 128
ZSTEPS = RPT // ZR
IDX_SRC, IDX_DST = 0, 1

_MM_BLOCK = 1000


def _mm_body(x_ref, w0_ref, w1_ref, o0_ref, o1_ref):
    x = x_ref[...]
    o0_ref[...] = jnp.dot(x, w0_ref[...], preferred_element_type=jnp.float32)
    o1_ref[...] = jnp.dot(x, w1_ref[...], preferred_element_type=jnp.float32)


def _matmul(x, W0, W1):
    grid = (N // _MM_BLOCK,)
    return pl.pallas_call(
        _mm_body,
        grid=grid,
        in_specs=[
            pl.BlockSpec((_MM_BLOCK, D), lambda i: (i, 0)),
            pl.BlockSpec((D, D), lambda i: (0, 0)),
            pl.BlockSpec((D, D), lambda i: (0, 0)),
        ],
        out_specs=[
            pl.BlockSpec((_MM_BLOCK, D), lambda i: (i, 0)),
            pl.BlockSpec((_MM_BLOCK, D), lambda i: (i, 0)),
        ],
        out_shape=[
            jax.ShapeDtypeStruct((N, D), jnp.float32),
            jax.ShapeDtypeStruct((N, D), jnp.float32),
        ],
    )(x, W0, W1)


def _epi_body(a0_ref, a1_ref, o_ref):
    o = jnp.maximum(a0_ref[...], 0.0) + jnp.maximum(a1_ref[...], 0.0)
    ss = jnp.sum(o * o, axis=1, keepdims=True)
    norm = jnp.maximum(jnp.sqrt(ss), 1e-12)
    o_ref[...] = o / norm


def _epilogue(a0, a1):
    blk = 1000
    grid = (N // blk,)
    return pl.pallas_call(
        _epi_body,
        grid=grid,
        in_specs=[
            pl.BlockSpec((blk, D), lambda i: (i, 0)),
            pl.BlockSpec((blk, D), lambda i: (i, 0)),
        ],
        out_specs=pl.BlockSpec((blk, D), lambda i: (i, 0)),
        out_shape=jax.ShapeDtypeStruct((N, D), jnp.float32),
    )(a0, a1)


_sc_mesh = plsc.VectorSubcoreMesh(core_axis_name="c", subcore_axis_name="s")


@functools.partial(
    pl.kernel,
    out_type=(
        jax.ShapeDtypeStruct((NPAD, D), jnp.float32),
        jax.ShapeDtypeStruct((NPAD, D), jnp.float32),
    ),
    mesh=_sc_mesh,
    scratch_types=[
        pltpu.VMEM_SHARED((NPAD, D), jnp.float32),
        [pltpu.VMEM((K, D), jnp.float32) for _ in range(2)],
        [pltpu.VMEM((2, NCH_SLAB, K), jnp.int32) for _ in range(2)],
        [pltpu.SemaphoreType.DMA for _ in range(2)],
        [pltpu.SemaphoreType.DMA for _ in range(2)],
        [pltpu.SemaphoreType.DMA for _ in range(2)],
    ],
)
def _sc_message_pass(xw0, idx0, xw1, idx1, out0, out1,
                     acc, rows, ibufs, gsems, ssems, isems):
    c = lax.axis_index("c")
    s = lax.axis_index("s")

    def _zero_body(i, carry):
        r = i // (D // 16)
        j = i % (D // 16)
        rows[0][r, pl.ds(j * 16, 16)] = jnp.zeros((16,), jnp.float32)
        return carry

    lax.fori_loop(0, ZR * (D // 16), _zero_body, 0)

    for t in range(ZSTEPS):
        pltpu.async_copy(rows[0], acc.at[pl.ds(s * RPT + t * ZR, ZR)],
                         isems[0])
    for t in range(ZSTEPS):
        pltpu.make_async_copy(rows[0], acc.at[pl.ds(s * RPT + t * ZR, ZR)],
                              isems[0]).wait()
    plsc.subcore_barrier()

    def _run_relation(xw, idx, out):
        def _gather(p, j, b):
            pltpu.async_copy(xw.at[ibufs[p].at[IDX_SRC].at[j]], rows[b],
                             gsems[b])

        def _wait_gather(p, j, b):
            pltpu.make_async_copy(xw.at[ibufs[p].at[IDX_SRC].at[j]], rows[b],
                                  gsems[b]).wait()

        def _scatter(p, j, b):
            pltpu.async_copy(rows[b], acc.at[ibufs[p].at[IDX_DST].at[j]],
                             ssems[b], add=True)

        def _wait_scatter(p, j, b):
            pltpu.make_async_copy(rows[b], acc.at[ibufs[p].at[IDX_DST].at[j]],
                                  ssems[b]).wait()

        def _prefetch(slab, p):
            pltpu.async_copy(idx.at[s].at[slab], ibufs[p], isems[p])

        def _wait_prefetch(slab, p):
            pltpu.make_async_copy(idx.at[s].at[slab], ibufs[p],
                                  isems[p]).wait()

        _prefetch(0, 0)

        def _body(i, carry):
            nz = i > 0

            @pl.when(nz)
            def _():
                _wait_gather(1, NCH_SLAB - 1, 1)
                _scatter(1, NCH_SLAB - 1, 1)

            _wait_prefetch(2 * i, 0)

            @pl.when(nz)
            def _():
                _wait_scatter(0, 0, 0)
            _gather(0, 0, 0)
            @pl.when(nz)
            def _():
                _wait_scatter(0, 0, 1)
            _gather(0, 1, 1)
            _prefetch(2 * i + 1, 1)

            _wait_gather(0, 0, 0)
            _scatter(0, 0, 0)
            _wait_scatter(0, 0, 0)
            _gather(0, 2, 0)
            _wait_gather(0, 1, 1)
            _scatter(0, 1, 1)
            _wait_scatter(0, 1, 1)
            _gather(0, 3, 1)
            _wait_gather(0, 2, 0)
            _scatter(0, 2, 0)

            _wait_gather(0, 3, 1)
            _scatter(0, 3, 1)
            _wait_prefetch(2 * i + 1, 1)
            _wait_scatter(0, 2, 0)
            _gather(1, 0, 0)
            _wait_scatter(0, 3, 1)
            _gather(1, 1, 1)
            nxt = jnp.minimum(2 * i + 2, NSLAB - 1)
            _prefetch(nxt, 0)

            _wait_gather(1, 0, 0)
            _scatter(1, 0, 0)
            _wait_scatter(1, 0, 0)
            _gather(1, 2, 0)
            _wait_gather(1, 1, 1)
            _scatter(1, 1, 1)
            _wait_scatter(1, 1, 1)
            _gather(1, 3, 1)
            _wait_gather(1, 2, 0)
            _scatter(1, 2, 0)
            return carry

        lax.fori_loop(0, NPAIR, _body, 0)

        _wait_gather(1, NCH_SLAB - 1, 1)
        _scatter(1, NCH_SLAB - 1, 1)
        _wait_scatter(0, 0, 0)
        _wait_scatter(0, 0, 1)
        _wait_prefetch(NSLAB - 1, 0)

        plsc.subcore_barrier()
        for t in range(ZSTEPS):
            base = s * RPT + t * ZR
            pltpu.async_copy(acc.at[pl.ds(base, ZR)],
                             out.at[pl.ds(base, ZR)], isems[1])
        for t in range(ZSTEPS):
            pltpu.make_async_copy(acc.at[pl.ds(t * ZR, ZR)],
                                  out.at[pl.ds(t * ZR, ZR)], isems[1]).wait()

    @pl.when(c == 0)
    def _():
        _run_relation(xw0, idx0, out0)

    @pl.when(c == 1)
    def _():
        _run_relation(xw1, idx1, out1)


def _pad_edges(ei):
    pad = EPAD - E
    src = jnp.concatenate(
        [ei[0], (jnp.arange(pad, dtype=jnp.int32) * 97) % N])
    dst = jnp.concatenate(
        [ei[1], N + (jnp.arange(pad, dtype=jnp.int32) % (NPAD - N))])
    src = src.reshape(NS, NSLAB, NCH_SLAB, K)
    dst = dst.reshape(NS, NSLAB, NCH_SLAB, K)
    return jnp.stack([src, dst], axis=2)


def kernel(inputs, edge_index0, edge_index1, W0, W1, dropout):
    xw0, xw1 = _matmul(inputs, W0, W1)
    idx0 = _pad_edges(edge_index0)
    idx1 = _pad_edges(edge_index1)
    acc0, acc1 = _sc_message_pass(xw0, idx0, xw1, idx1)
    return _epilogue(acc0, acc1)


# speedup vs baseline: 11.3071x; 1.0182x over previous
import functools

import jax
import jax.numpy as jnp
from jax import lax
from jax.experimental import pallas as pl
from jax.experimental.pallas import tpu as pltpu
from jax.experimental.pallas import tpu_sc as plsc

N = 10000
E = 320000
D = 128

NS = 16
K = 112
CPT = 180
NCH_SLAB = 6
NSLAB = CPT // NCH_SLAB
NPAIR = NSLAB // 2
EPAD = NS * CPT * K
NPAD = 10240
RPT = NPAD // NS
ZR = 80
ZSTEPS = RPT // ZR
WR = 128
WSTEPS = RPT // WR
IDX_SRC, IDX_DST = 0, 1

_MM_BLOCK = 1000


def _mm_body(x_ref, w0_ref, w1_ref, o0_ref, o1_ref):
    x = x_ref[...]
    o0_ref[...] = jnp.dot(x, w0_ref[...], preferred_element_type=jnp.float32)
    o1_ref[...] = jnp.dot(x, w1_ref[...], preferred_element_type=jnp.float32)


def _matmul(x, W0, W1):
    grid = (N // _MM_BLOCK,)
    return pl.pallas_call(
        _mm_body,
        grid=grid,
        in_specs=[
            pl.BlockSpec((_MM_BLOCK, D), lambda i: (i, 0)),
            pl.BlockSpec((D, D), lambda i: (0, 0)),
            pl.BlockSpec((D, D), lambda i: (0, 0)),
        ],
        out_specs=[
            pl.BlockSpec((_MM_BLOCK, D), lambda i: (i, 0)),
            pl.BlockSpec((_MM_BLOCK, D), lambda i: (i, 0)),
        ],
        out_shape=[
            jax.ShapeDtypeStruct((N, D), jnp.float32),
            jax.ShapeDtypeStruct((N, D), jnp.float32),
        ],
    )(x, W0, W1)


def _epi_body(a0_ref, a1_ref, o_ref):
    o = jnp.maximum(a0_ref[...], 0.0) + jnp.maximum(a1_ref[...], 0.0)
    ss = jnp.sum(o * o, axis=1, keepdims=True)
    norm = jnp.maximum(jnp.sqrt(ss), 1e-12)
    o_ref[...] = o / norm


def _epilogue(a0, a1):
    blk = 1000
    grid = (N // blk,)
    return pl.pallas_call(
        _epi_body,
        grid=grid,
        in_specs=[
            pl.BlockSpec((blk, D), lambda i: (i, 0)),
            pl.BlockSpec((blk, D), lambda i: (i, 0)),
        ],
        out_specs=pl.BlockSpec((blk, D), lambda i: (i, 0)),
        out_shape=jax.ShapeDtypeStruct((N, D), jnp.float32),
    )(a0, a1)


_sc_mesh = plsc.VectorSubcoreMesh(core_axis_name="c", subcore_axis_name="s")


@functools.partial(
    pl.kernel,
    out_type=(
        jax.ShapeDtypeStruct((NPAD, D), jnp.float32),
        jax.ShapeDtypeStruct((NPAD, D), jnp.float32),
    ),
    mesh=_sc_mesh,
    scratch_types=[
        pltpu.VMEM_SHARED((NPAD, D), jnp.float32),
        [pltpu.VMEM((K, D), jnp.float32) for _ in range(3)],
        [pltpu.VMEM((2, NCH_SLAB, K), jnp.int32) for _ in range(2)],
        [pltpu.SemaphoreType.DMA for _ in range(3)],
        [pltpu.SemaphoreType.DMA for _ in range(3)],
        [pltpu.SemaphoreType.DMA for _ in range(2)],
    ],
)
def _sc_message_pass(xw0, idx0, xw1, idx1, out0, out1,
                     acc, rows, ibufs, gsems, ssems, isems):
    c = lax.axis_index("c")
    s = lax.axis_index("s")

    def _zero_body(i, carry):
        r = i // (D // 16)
        j = i % (D // 16)
        rows[0][r, pl.ds(j * 16, 16)] = jnp.zeros((16,), jnp.float32)
        return carry

    lax.fori_loop(0, ZR * (D // 16), _zero_body, 0)

    zsrc = rows[0].at[pl.ds(0, ZR)]
    for t in range(ZSTEPS):
        pltpu.async_copy(zsrc, acc.at[pl.ds(s * RPT + t * ZR, ZR)], isems[0])
    for t in range(ZSTEPS):
        pltpu.make_async_copy(zsrc, acc.at[pl.ds(s * RPT + t * ZR, ZR)],
                              isems[0]).wait()
    plsc.subcore_barrier()

    def _run_relation(xw, idx, out):
        def _gather(p, j, b):
            pltpu.async_copy(xw.at[ibufs[p].at[IDX_SRC].at[j]], rows[b],
                             gsems[b])

        def _wait_gather(p, j, b):
            pltpu.make_async_copy(xw.at[ibufs[p].at[IDX_SRC].at[j]], rows[b],
                                  gsems[b]).wait()

        def _scatter(p, j, b):
            pltpu.async_copy(rows[b], acc.at[ibufs[p].at[IDX_DST].at[j]],
                             ssems[b], add=True)

        def _wait_scatter(b):
            pltpu.make_async_copy(rows[b], acc.at[ibufs[0].at[IDX_DST].at[0]],
                                  ssems[b]).wait()

        def _prefetch(slab, p):
            pltpu.async_copy(idx.at[s].at[slab], ibufs[p], isems[p])

        def _wait_prefetch(slab, p):
            pltpu.make_async_copy(idx.at[s].at[slab], ibufs[p],
                                  isems[p]).wait()

        _prefetch(0, 0)

        def _body(i, carry):
            nz = i > 0

            @pl.when(nz)
            def _():
                _wait_gather(1, 5, 2)
                _scatter(1, 5, 2)

            _wait_prefetch(2 * i, 0)

            @pl.when(nz)
            def _():
                _wait_scatter(0)
            _gather(0, 0, 0)
            @pl.when(nz)
            def _():
                _wait_scatter(1)
            _gather(0, 1, 1)
            _wait_gather(0, 0, 0)
            _scatter(0, 0, 0)
            @pl.when(nz)
            def _():
                _wait_scatter(2)
            _gather(0, 2, 2)
            _prefetch(2 * i + 1, 1)
            _wait_gather(0, 1, 1)
            _scatter(0, 1, 1)
            _wait_scatter(0)
            _gather(0, 3, 0)
            _wait_gather(0, 2, 2)
            _scatter(0, 2, 2)
            _wait_scatter(1)
            _gather(0, 4, 1)
            _wait_gather(0, 3, 0)
            _scatter(0, 3, 0)
            _wait_scatter(2)
            _gather(0, 5, 2)
            _wait_gather(0, 4, 1)
            _scatter(0, 4, 1)
            _wait_prefetch(2 * i + 1, 1)
            _wait_scatter(0)
            _gather(1, 0, 0)
            _wait_gather(0, 5, 2)
            _scatter(0, 5, 2)
            _wait_scatter(1)
            _gather(1, 1, 1)
            _wait_gather(1, 0, 0)
            _scatter(1, 0, 0)
            _wait_scatter(2)
            _gather(1, 2, 2)
            nxt = jnp.minimum(2 * i + 2, NSLAB - 1)
            _prefetch(nxt, 0)
            _wait_gather(1, 1, 1)
            _scatter(1, 1, 1)
            _wait_scatter(0)
            _gather(1, 3, 0)
            _wait_gather(1, 2, 2)
            _scatter(1, 2, 2)
            _wait_scatter(1)
            _gather(1, 4, 1)
            _wait_gather(1, 3, 0)
            _scatter(1, 3, 0)
            _wait_scatter(2)
            _gather(1, 5, 2)
            _wait_gather(1, 4, 1)
            _scatter(1, 4, 1)
            return carry

        lax.fori_loop(0, NPAIR, _body, 0)

        _wait_gather(1, 5, 2)
        _scatter(1, 5, 2)
        _wait_scatter(0)
        _wait_scatter(1)
        _wait_scatter(2)
        _wait_prefetch(NSLAB - 1, 0)

        plsc.subcore_barrier()
        for t in range(WSTEPS):
            base = s * RPT + t * WR
            pltpu.async_copy(acc.at[pl.ds(base, WR)],
                             out.at[pl.ds(base, WR)], isems[1])
        for t in range(WSTEPS):
            pltpu.make_async_copy(acc.at[pl.ds(t * WR, WR)],
                                  out.at[pl.ds(t * WR, WR)], isems[1]).wait()

    @pl.when(c == 0)
    def _():
        _run_relation(xw0, idx0, out0)

    @pl.when(c == 1)
    def _():
        _run_relation(xw1, idx1, out1)


def _pad_edges(ei):
    pad = EPAD - E
    src = jnp.concatenate(
        [ei[0], (jnp.arange(pad, dtype=jnp.int32) * 97) % N])
    dst = jnp.concatenate(
        [ei[1], N + (jnp.arange(pad, dtype=jnp.int32) % (NPAD - N))])
    src = src.reshape(NS, NSLAB, NCH_SLAB, K)
    dst = dst.reshape(NS, NSLAB, NCH_SLAB, K)
    return jnp.stack([src, dst], axis=2)


def kernel(inputs, edge_index0, edge_index1, W0, W1, dropout):
    xw0, xw1 = _matmul(inputs, W0, W1)
    idx0 = _pad_edges(edge_index0)
    idx1 = _pad_edges(edge_index1)
    acc0, acc1 = _sc_message_pass(xw0, idx0, xw1, idx1)
    return _epilogue(acc0, acc1)


# speedup vs baseline: 11.9138x; 1.0537x over previous
import functools

import jax
import jax.numpy as jnp
import numpy as np
from jax import lax
from jax.experimental import pallas as pl
from jax.experimental.pallas import tpu as pltpu
from jax.experimental.pallas import tpu_sc as plsc

N = 10000
E = 320000
D = 128

NS = 16
K = 128
CPT = 160
EPTP = CPT * K
EPAD = NS * EPTP
NPAD = 10240
RPT = NPAD // NS
ZR = 128
ZSTEPS = RPT // ZR

_PAD = EPAD - E
_SRC_PAD = np.asarray((np.arange(_PAD) * 97) % N, np.int32)
_DST_PAD = np.asarray(N + (np.arange(_PAD) % (NPAD - N)), np.int32)

_MM_BLOCK = 1000


def _mm_body(x_ref, w0_ref, w1_ref, o0_ref, o1_ref):
    x = x_ref[...]
    o0_ref[...] = jnp.dot(x, w0_ref[...], preferred_element_type=jnp.float32)
    o1_ref[...] = jnp.dot(x, w1_ref[...], preferred_element_type=jnp.float32)


def _matmul(x, W0, W1):
    grid = (N // _MM_BLOCK,)
    return pl.pallas_call(
        _mm_body,
        grid=grid,
        in_specs=[
            pl.BlockSpec((_MM_BLOCK, D), lambda i: (i, 0)),
            pl.BlockSpec((D, D), lambda i: (0, 0)),
            pl.BlockSpec((D, D), lambda i: (0, 0)),
        ],
        out_specs=[
            pl.BlockSpec((_MM_BLOCK, D), lambda i: (i, 0)),
            pl.BlockSpec((_MM_BLOCK, D), lambda i: (i, 0)),
        ],
        out_shape=[
            jax.ShapeDtypeStruct((N, D), jnp.float32),
            jax.ShapeDtypeStruct((N, D), jnp.float32),
        ],
    )(x, W0, W1)


def _epi_body(a0_ref, a1_ref, o_ref):
    o = jnp.maximum(a0_ref[...], 0.0) + jnp.maximum(a1_ref[...], 0.0)
    ss = jnp.sum(o * o, axis=1, keepdims=True)
    norm = jnp.maximum(jnp.sqrt(ss), 1e-12)
    o_ref[...] = o / norm


def _epilogue(a0, a1):
    blk = 1000
    grid = (N // blk,)
    return pl.pallas_call(
        _epi_body,
        grid=grid,
        in_specs=[
            pl.BlockSpec((blk, D), lambda i: (i, 0)),
            pl.BlockSpec((blk, D), lambda i: (i, 0)),
        ],
        out_specs=pl.BlockSpec((blk, D), lambda i: (i, 0)),
        out_shape=jax.ShapeDtypeStruct((N, D), jnp.float32),
    )(a0, a1)


_sc_mesh = plsc.VectorSubcoreMesh(core_axis_name="c", subcore_axis_name="s")


@functools.partial(
    pl.kernel,
    out_type=(
        jax.ShapeDtypeStruct((NPAD, D), jnp.float32),
        jax.ShapeDtypeStruct((NPAD, D), jnp.float32),
    ),
    mesh=_sc_mesh,
    scratch_types=[
        pltpu.VMEM_SHARED((NPAD, D), jnp.float32),
        [pltpu.VMEM((K, D), jnp.float32) for _ in range(2)],
        [pltpu.VMEM((K,), jnp.int32) for _ in range(4)],
        [pltpu.VMEM((K,), jnp.int32) for _ in range(4)],
        [pltpu.SemaphoreType.DMA for _ in range(2)],
        [pltpu.SemaphoreType.DMA for _ in range(2)],
        [pltpu.SemaphoreType.DMA for _ in range(4)],
    ],
)
def _sc_message_pass(xw0, src0, dst0, xw1, src1, dst1, out0, out1,
                     acc, rows, sbuf, dbuf, gsems, ssems, isems):
    c = lax.axis_index("c")
    s = lax.axis_index("s")

    def _zero_body(i, carry):
        r = i // (D // 16)
        j = i % (D // 16)
        rows[0][r, pl.ds(j * 16, 16)] = jnp.zeros((16,), jnp.float32)
        return carry

    lax.fori_loop(0, ZR * (D // 16), _zero_body, 0)

    for t in range(ZSTEPS):
        pltpu.async_copy(rows[0], acc.at[pl.ds(s * RPT + t * ZR, ZR)],
                         isems[0])
    for t in range(ZSTEPS):
        pltpu.make_async_copy(rows[0], acc.at[pl.ds(s * RPT + t * ZR, ZR)],
                              isems[0]).wait()
    plsc.subcore_barrier()

    def _run_relation(xw, srcp, dstp, out):
        base = s * EPTP

        def _idx_load(e, slot):
            off = base + e * K
            pltpu.async_copy(srcp.at[pl.ds(off, K)], sbuf[slot], isems[slot])
            pltpu.async_copy(dstp.at[pl.ds(off, K)], dbuf[slot], isems[slot])

        def _idx_wait(slot):
            pltpu.make_async_copy(srcp.at[pl.ds(base, K)], sbuf[slot],
                                  isems[slot]).wait()
            pltpu.make_async_copy(dstp.at[pl.ds(base, K)], dbuf[slot],
                                  isems[slot]).wait()

        def _gather(slot, b):
            pltpu.async_copy(xw.at[sbuf[slot]], rows[b], gsems[b])

        def _gwait(slot, b):
            pltpu.make_async_copy(xw.at[sbuf[slot]], rows[b], gsems[b]).wait()

        def _scatter(slot, b):
            pltpu.async_copy(rows[b], acc.at[dbuf[slot]], ssems[b], add=True)

        def _swait(b):
            pltpu.make_async_copy(rows[b], acc.at[dbuf[0]], ssems[b]).wait()

        _idx_load(0, 0)
        _idx_load(1, 1)

        def _body(i, carry):
            nz = i > 0
            for u in range(4):
                b = u % 2
                pb = (u + 1) % 2
                if u < 2:
                    @pl.when(nz)
                    def _():
                        _swait(b)
                else:
                    _swait(b)
                e = jnp.minimum(4 * i + u + 2, CPT - 1)
                _idx_load(e, (u + 2) % 4)
                _idx_wait(u)
                _gather(u, b)
                if u == 0:
                    @pl.when(nz)
                    def _():
                        _gwait(3, pb)
                        _scatter(3, pb)
                else:
                    _gwait(u - 1, pb)
                    _scatter(u - 1, pb)
            return carry

        lax.fori_loop(0, CPT // 4, _body, 0)

        _gwait(3, 1)
        _scatter(3, 1)
        _swait(0)
        _swait(1)
        _idx_wait(0)
        _idx_wait(1)

        plsc.subcore_barrier()
        for t in range(ZSTEPS):
            rbase = s * RPT + t * ZR
            pltpu.async_copy(acc.at[pl.ds(rbase, ZR)],
                             out.at[pl.ds(rbase, ZR)], isems[1])
        for t in range(ZSTEPS):
            pltpu.make_async_copy(acc.at[pl.ds(t * ZR, ZR)],
                                  out.at[pl.ds(t * ZR, ZR)], isems[1]).wait()

    @pl.when(c == 0)
    def _():
        _run_relation(xw0, src0, dst0, out0)

    @pl.when(c == 1)
    def _():
        _run_relation(xw1, src1, dst1, out1)


def kernel(inputs, edge_index0, edge_index1, W0, W1, dropout):
    xw0, xw1 = _matmul(inputs, W0, W1)
    sp = jnp.asarray(_SRC_PAD)
    dp = jnp.asarray(_DST_PAD)
    src0 = jnp.concatenate([edge_index0[0], sp])
    dst0 = jnp.concatenate([edge_index0[1], dp])
    src1 = jnp.concatenate([edge_index1[0], sp])
    dst1 = jnp.concatenate([edge_index1[1], dp])
    acc0, acc1 = _sc_message_pass(xw0, src0, dst0, xw1, src1, dst1)
    return _epilogue(acc0, acc1)


# speedup vs baseline: 13.1520x; 1.1039x over previous
import functools

import jax
import jax.numpy as jnp
import numpy as np
from jax import lax
from jax.experimental import pallas as pl
from jax.experimental.pallas import tpu as pltpu
from jax.experimental.pallas import tpu_sc as plsc

N = 10000
E = 320000
D = 128

NS = 16
K = 128
CPT = 160
EPTP = CPT * K
EPAD = NS * EPTP
NPAD = 10240
RPT = NPAD // NS
ZR = 128
ZSTEPS = RPT // ZR

_PAD = EPAD - E
_SRC_PAD = np.asarray((np.arange(_PAD) * 97) % N, np.int32)
_DST_PAD = np.asarray(N + (np.arange(_PAD) % (NPAD - N)), np.int32)

_MM_BLOCK = 1000
_GRID = N // _MM_BLOCK
_EBLK = EPAD // _GRID
_TAIL = E - (_GRID - 1) * _EBLK


def _mm_body(x_ref, w0_ref, w1_ref, e0_ref, e1_ref, sp_ref, dp_ref,
             o0_ref, o1_ref, s0_ref, d0_ref, s1_ref, d1_ref):
    i = pl.program_id(0)
    x = x_ref[...]
    o0_ref[...] = jnp.dot(x, w0_ref[...], preferred_element_type=jnp.float32)
    o1_ref[...] = jnp.dot(x, w1_ref[...], preferred_element_type=jnp.float32)
    s0_ref[...] = e0_ref[0, :]
    d0_ref[...] = e0_ref[1, :]
    s1_ref[...] = e1_ref[0, :]
    d1_ref[...] = e1_ref[1, :]

    @pl.when(i == _GRID - 1)
    def _():
        s0_ref[pl.ds(_TAIL, _PAD)] = sp_ref[...]
        d0_ref[pl.ds(_TAIL, _PAD)] = dp_ref[...]
        s1_ref[pl.ds(_TAIL, _PAD)] = sp_ref[...]
        d1_ref[pl.ds(_TAIL, _PAD)] = dp_ref[...]


def _matmul(x, W0, W1, ei0, ei1, spad, dpad):
    grid = (_GRID,)
    return pl.pallas_call(
        _mm_body,
        grid=grid,
        in_specs=[
            pl.BlockSpec((_MM_BLOCK, D), lambda i: (i, 0)),
            pl.BlockSpec((D, D), lambda i: (0, 0)),
            pl.BlockSpec((D, D), lambda i: (0, 0)),
            pl.BlockSpec((2, _EBLK), lambda i: (0, i)),
            pl.BlockSpec((2, _EBLK), lambda i: (0, i)),
            pl.BlockSpec((_PAD,), lambda i: (0,)),
            pl.BlockSpec((_PAD,), lambda i: (0,)),
        ],
        out_specs=[
            pl.BlockSpec((_MM_BLOCK, D), lambda i: (i, 0)),
            pl.BlockSpec((_MM_BLOCK, D), lambda i: (i, 0)),
            pl.BlockSpec((_EBLK,), lambda i: (i,)),
            pl.BlockSpec((_EBLK,), lambda i: (i,)),
            pl.BlockSpec((_EBLK,), lambda i: (i,)),
            pl.BlockSpec((_EBLK,), lambda i: (i,)),
        ],
        out_shape=[
            jax.ShapeDtypeStruct((N, D), jnp.float32),
            jax.ShapeDtypeStruct((N, D), jnp.float32),
            jax.ShapeDtypeStruct((EPAD,), jnp.int32),
            jax.ShapeDtypeStruct((EPAD,), jnp.int32),
            jax.ShapeDtypeStruct((EPAD,), jnp.int32),
            jax.ShapeDtypeStruct((EPAD,), jnp.int32),
        ],
    )(x, W0, W1, ei0, ei1, spad, dpad)


def _epi_body(a0_ref, a1_ref, o_ref):
    o = jnp.maximum(a0_ref[...], 0.0) + jnp.maximum(a1_ref[...], 0.0)
    ss = jnp.sum(o * o, axis=1, keepdims=True)
    norm = jnp.maximum(jnp.sqrt(ss), 1e-12)
    o_ref[...] = o / norm


def _epilogue(a0, a1):
    blk = 1000
    grid = (N // blk,)
    return pl.pallas_call(
        _epi_body,
        grid=grid,
        in_specs=[
            pl.BlockSpec((blk, D), lambda i: (i, 0)),
            pl.BlockSpec((blk, D), lambda i: (i, 0)),
        ],
        out_specs=pl.BlockSpec((blk, D), lambda i: (i, 0)),
        out_shape=jax.ShapeDtypeStruct((N, D), jnp.float32),
    )(a0, a1)


_sc_mesh = plsc.VectorSubcoreMesh(core_axis_name="c", subcore_axis_name="s")


@functools.partial(
    pl.kernel,
    out_type=(
        jax.ShapeDtypeStruct((NPAD, D), jnp.float32),
        jax.ShapeDtypeStruct((NPAD, D), jnp.float32),
    ),
    mesh=_sc_mesh,
    scratch_types=[
        pltpu.VMEM_SHARED((NPAD, D), jnp.float32),
        [pltpu.VMEM((K, D), jnp.float32) for _ in range(2)],
        [pltpu.VMEM((K,), jnp.int32) for _ in range(4)],
        [pltpu.VMEM((K,), jnp.int32) for _ in range(4)],
        [pltpu.SemaphoreType.DMA for _ in range(2)],
        [pltpu.SemaphoreType.DMA for _ in range(2)],
        [pltpu.SemaphoreType.DMA for _ in range(4)],
    ],
)
def _sc_message_pass(xw0, src0, dst0, xw1, src1, dst1, out0, out1,
                     acc, rows, sbuf, dbuf, gsems, ssems, isems):
    c = lax.axis_index("c")
    s = lax.axis_index("s")

    def _zero_body(i, carry):
        r = i // (D // 16)
        j = i % (D // 16)
        rows[0][r, pl.ds(j * 16, 16)] = jnp.zeros((16,), jnp.float32)
        return carry

    lax.fori_loop(0, ZR * (D // 16), _zero_body, 0)

    for t in range(ZSTEPS):
        pltpu.async_copy(rows[0], acc.at[pl.ds(s * RPT + t * ZR, ZR)],
                         isems[0])
    for t in range(ZSTEPS):
        pltpu.make_async_copy(rows[0], acc.at[pl.ds(s * RPT + t * ZR, ZR)],
                              isems[0]).wait()
    plsc.subcore_barrier()

    def _run_relation(xw, srcp, dstp, out):
        base = s * EPTP

        def _idx_load(e, slot):
            off = base + e * K
            pltpu.async_copy(srcp.at[pl.ds(off, K)], sbuf[slot], isems[slot])
            pltpu.async_copy(dstp.at[pl.ds(off, K)], dbuf[slot], isems[slot])

        def _idx_wait(slot):
            pltpu.make_async_copy(srcp.at[pl.ds(base, K)], sbuf[slot],
                                  isems[slot]).wait()
            pltpu.make_async_copy(dstp.at[pl.ds(base, K)], dbuf[slot],
                                  isems[slot]).wait()

        def _gather(slot, b):
            pltpu.async_copy(xw.at[sbuf[slot]], rows[b], gsems[b])

        def _gwait(slot, b):
            pltpu.make_async_copy(xw.at[sbuf[slot]], rows[b], gsems[b]).wait()

        def _scatter(slot, b):
            pltpu.async_copy(rows[b], acc.at[dbuf[slot]], ssems[b], add=True)

        def _swait(b):
            pltpu.make_async_copy(rows[b], acc.at[dbuf[0]], ssems[b]).wait()

        _idx_load(0, 0)
        _idx_load(1, 1)

        def _body(i, carry):
            nz = i > 0
            for u in range(4):
                b = u % 2
                pb = (u + 1) % 2
                if u < 2:
                    @pl.when(nz)
                    def _():
                        _swait(b)
                else:
                    _swait(b)
                e = jnp.minimum(4 * i + u + 2, CPT - 1)
                _idx_load(e, (u + 2) % 4)
                _idx_wait(u)
                _gather(u, b)
                if u == 0:
                    @pl.when(nz)
                    def _():
                        _gwait(3, pb)
                        _scatter(3, pb)
                else:
                    _gwait(u - 1, pb)
                    _scatter(u - 1, pb)
            return carry

        lax.fori_loop(0, CPT // 4, _body, 0)

        _gwait(3, 1)
        _scatter(3, 1)
        _swait(0)
        _swait(1)
        _idx_wait(0)
        _idx_wait(1)

        plsc.subcore_barrier()
        for t in range(ZSTEPS):
            rbase = s * RPT + t * ZR
            pltpu.async_copy(acc.at[pl.ds(rbase, ZR)],
                             out.at[pl.ds(rbase, ZR)], isems[1])
        for t in range(ZSTEPS):
            pltpu.make_async_copy(acc.at[pl.ds(t * ZR, ZR)],
                                  out.at[pl.ds(t * ZR, ZR)], isems[1]).wait()

    @pl.when(c == 0)
    def _():
        _run_relation(xw0, src0, dst0, out0)

    @pl.when(c == 1)
    def _():
        _run_relation(xw1, src1, dst1, out1)


def kernel(inputs, edge_index0, edge_index1, W0, W1, dropout):
    sp = jnp.asarray(_SRC_PAD)
    dp = jnp.asarray(_DST_PAD)
    xw0, xw1, src0, dst0, src1, dst1 = _matmul(
        inputs, W0, W1, edge_index0, edge_index1, sp, dp)
    acc0, acc1 = _sc_message_pass(xw0, src0, dst0, xw1, src1, dst1)
    return _epilogue(acc0, acc1)


# speedup vs baseline: 13.2359x; 1.0064x over previous
import functools

import jax
import jax.numpy as jnp
import numpy as np
from jax import lax
from jax.experimental import pallas as pl
from jax.experimental.pallas import tpu as pltpu
from jax.experimental.pallas import tpu_sc as plsc

N = 10000
E = 320000
D = 128

NS = 16
K = 128
CPT = 160
EPTP = CPT * K
EPAD = NS * EPTP
NPAD = 10240
RPT = NPAD // NS
ZR = 128
ZSTEPS = RPT // ZR

_PAD = EPAD - E
_SRC_PAD = np.asarray((np.arange(_PAD) * 97) % N, np.int32)
_DST_PAD = np.asarray(N + (np.arange(_PAD) % (NPAD - N)), np.int32)

_MM_BLOCK = 1000
_GRID = N // _MM_BLOCK
_EBLK = EPAD // _GRID
_TAIL = E - (_GRID - 1) * _EBLK


def _mm_body(x_ref, w0_ref, w1_ref, e0_ref, e1_ref, sp_ref, dp_ref,
             o0_ref, o1_ref, s0_ref, d0_ref, s1_ref, d1_ref):
    i = pl.program_id(0)
    x = x_ref[...].astype(jnp.bfloat16)
    w0 = w0_ref[...].astype(jnp.bfloat16)
    w1 = w1_ref[...].astype(jnp.bfloat16)
    o0_ref[...] = jnp.dot(x, w0, preferred_element_type=jnp.float32)
    o1_ref[...] = jnp.dot(x, w1, preferred_element_type=jnp.float32)
    s0_ref[...] = e0_ref[0, :]
    d0_ref[...] = e0_ref[1, :]
    s1_ref[...] = e1_ref[0, :]
    d1_ref[...] = e1_ref[1, :]

    @pl.when(i == _GRID - 1)
    def _():
        s0_ref[pl.ds(_TAIL, _PAD)] = sp_ref[...]
        d0_ref[pl.ds(_TAIL, _PAD)] = dp_ref[...]
        s1_ref[pl.ds(_TAIL, _PAD)] = sp_ref[...]
        d1_ref[pl.ds(_TAIL, _PAD)] = dp_ref[...]


def _matmul(x, W0, W1, ei0, ei1, spad, dpad):
    grid = (_GRID,)
    return pl.pallas_call(
        _mm_body,
        grid=grid,
        in_specs=[
            pl.BlockSpec((_MM_BLOCK, D), lambda i: (i, 0)),
            pl.BlockSpec((D, D), lambda i: (0, 0)),
            pl.BlockSpec((D, D), lambda i: (0, 0)),
            pl.BlockSpec((2, _EBLK), lambda i: (0, i)),
            pl.BlockSpec((2, _EBLK), lambda i: (0, i)),
            pl.BlockSpec((_PAD,), lambda i: (0,)),
            pl.BlockSpec((_PAD,), lambda i: (0,)),
        ],
        out_specs=[
            pl.BlockSpec((_MM_BLOCK, D), lambda i: (i, 0)),
            pl.BlockSpec((_MM_BLOCK, D), lambda i: (i, 0)),
            pl.BlockSpec((_EBLK,), lambda i: (i,)),
            pl.BlockSpec((_EBLK,), lambda i: (i,)),
            pl.BlockSpec((_EBLK,), lambda i: (i,)),
            pl.BlockSpec((_EBLK,), lambda i: (i,)),
        ],
        out_shape=[
            jax.ShapeDtypeStruct((N, D), jnp.float32),
            jax.ShapeDtypeStruct((N, D), jnp.float32),
            jax.ShapeDtypeStruct((EPAD,), jnp.int32),
            jax.ShapeDtypeStruct((EPAD,), jnp.int32),
            jax.ShapeDtypeStruct((EPAD,), jnp.int32),
            jax.ShapeDtypeStruct((EPAD,), jnp.int32),
        ],
    )(x, W0, W1, ei0, ei1, spad, dpad)


def _epi_body(a0_ref, a1_ref, o_ref):
    o = jnp.maximum(a0_ref[...], 0.0) + jnp.maximum(a1_ref[...], 0.0)
    ss = jnp.sum(o * o, axis=1, keepdims=True)
    norm = jnp.maximum(jnp.sqrt(ss), 1e-12)
    o_ref[...] = o / norm


def _epilogue(a0, a1):
    blk = 1000
    grid = (N // blk,)
    return pl.pallas_call(
        _epi_body,
        grid=grid,
        in_specs=[
            pl.BlockSpec((blk, D), lambda i: (i, 0)),
            pl.BlockSpec((blk, D), lambda i: (i, 0)),
        ],
        out_specs=pl.BlockSpec((blk, D), lambda i: (i, 0)),
        out_shape=jax.ShapeDtypeStruct((N, D), jnp.float32),
    )(a0, a1)


_sc_mesh = plsc.VectorSubcoreMesh(core_axis_name="c", subcore_axis_name="s")


@functools.partial(
    pl.kernel,
    out_type=(
        jax.ShapeDtypeStruct((NPAD, D), jnp.float32),
        jax.ShapeDtypeStruct((NPAD, D), jnp.float32),
    ),
    mesh=_sc_mesh,
    scratch_types=[
        pltpu.VMEM_SHARED((NPAD, D), jnp.float32),
        [pltpu.VMEM((K, D), jnp.float32) for _ in range(2)],
        [pltpu.VMEM((K,), jnp.int32) for _ in range(4)],
        [pltpu.VMEM((K,), jnp.int32) for _ in range(4)],
        [pltpu.SemaphoreType.DMA for _ in range(2)],
        [pltpu.SemaphoreType.DMA for _ in range(2)],
        [pltpu.SemaphoreType.DMA for _ in range(4)],
    ],
)
def _sc_message_pass(xw0, src0, dst0, xw1, src1, dst1, out0, out1,
                     acc, rows, sbuf, dbuf, gsems, ssems, isems):
    c = lax.axis_index("c")
    s = lax.axis_index("s")

    def _zero_body(i, carry):
        r = i // (D // 16)
        j = i % (D // 16)
        rows[0][r, pl.ds(j * 16, 16)] = jnp.zeros((16,), jnp.float32)
        return carry

    lax.fori_loop(0, ZR * (D // 16), _zero_body, 0)

    for t in range(ZSTEPS):
        pltpu.async_copy(rows[0], acc.at[pl.ds(s * RPT + t * ZR, ZR)],
                         isems[0])
    for t in range(ZSTEPS):
        pltpu.make_async_copy(rows[0], acc.at[pl.ds(s * RPT + t * ZR, ZR)],
                              isems[0]).wait()
    plsc.subcore_barrier()

    def _run_relation(xw, srcp, dstp, out):
        base = s * EPTP

        def _idx_load(e, slot):
            off = base + e * K
            pltpu.async_copy(srcp.at[pl.ds(off, K)], sbuf[slot], isems[slot])
            pltpu.async_copy(dstp.at[pl.ds(off, K)], dbuf[slot], isems[slot])

        def _idx_wait(slot):
            pltpu.make_async_copy(srcp.at[pl.ds(base, K)], sbuf[slot],
                                  isems[slot]).wait()
            pltpu.make_async_copy(dstp.at[pl.ds(base, K)], dbuf[slot],
                                  isems[slot]).wait()

        def _gather(slot, b):
            pltpu.async_copy(xw.at[sbuf[slot]], rows[b], gsems[b])

        def _gwait(slot, b):
            pltpu.make_async_copy(xw.at[sbuf[slot]], rows[b], gsems[b]).wait()

        def _scatter(slot, b):
            pltpu.async_copy(rows[b], acc.at[dbuf[slot]], ssems[b], add=True)

        def _swait(b):
            pltpu.make_async_copy(rows[b], acc.at[dbuf[0]], ssems[b]).wait()

        _idx_load(0, 0)
        _idx_load(1, 1)

        def _body(i, carry):
            nz = i > 0
            for u in range(4):
                b = u % 2
                pb = (u + 1) % 2
                if u < 2:
                    @pl.when(nz)
                    def _():
                        _swait(b)
                else:
                    _swait(b)
                e = jnp.minimum(4 * i + u + 2, CPT - 1)
                _idx_load(e, (u + 2) % 4)
                _idx_wait(u)
                _gather(u, b)
                if u == 0:
                    @pl.when(nz)
                    def _():
                        _gwait(3, pb)
                        _scatter(3, pb)
                else:
                    _gwait(u - 1, pb)
                    _scatter(u - 1, pb)
            return carry

        lax.fori_loop(0, CPT // 4, _body, 0)

        _gwait(3, 1)
        _scatter(3, 1)
        _swait(0)
        _swait(1)
        _idx_wait(0)
        _idx_wait(1)

        plsc.subcore_barrier()
        for t in range(ZSTEPS):
            rbase = s * RPT + t * ZR
            pltpu.async_copy(acc.at[pl.ds(rbase, ZR)],
                             out.at[pl.ds(rbase, ZR)], isems[1])
        for t in range(ZSTEPS):
            pltpu.make_async_copy(acc.at[pl.ds(t * ZR, ZR)],
                                  out.at[pl.ds(t * ZR, ZR)], isems[1]).wait()

    @pl.when(c == 0)
    def _():
        _run_relation(xw0, src0, dst0, out0)

    @pl.when(c == 1)
    def _():
        _run_relation(xw1, src1, dst1, out1)


def kernel(inputs, edge_index0, edge_index1, W0, W1, dropout):
    sp = jnp.asarray(_SRC_PAD)
    dp = jnp.asarray(_DST_PAD)
    xw0, xw1, src0, dst0, src1, dst1 = _matmul(
        inputs, W0, W1, edge_index0, edge_index1, sp, dp)
    acc0, acc1 = _sc_message_pass(xw0, src0, dst0, xw1, src1, dst1)
    return _epilogue(acc0, acc1)
